# baseline (speedup 1.0000x reference)
import functools

import jax
import jax.numpy as jnp
from jax import lax
from jax.experimental import pallas as pl
from jax.experimental.pallas import tpu as pltpu
from jax.experimental.pallas import tpu_sc as plsc

B, S, D, M = 16, 4096, 768, 4
CS = 512
NC = (S + 1 + CS - 1) // CS
QP = 8
SCALE = D ** -0.5

FH = D // 2
NFB = FH // 16
NR = 128


def _tree3(vals, op):
    while len(vals) > 1:
        vals = [op(vals[i], vals[i + 1]) for i in range(0, len(vals) - 1, 2)] \
            + ([vals[-1]] if len(vals) % 2 else [])
    return vals[0]


def _sc_pool_body(lens_hbm, tokens_hbm, out_hbm, lens_v, buf, sum_v, max_v,
                  min_v, sem0, sem1):
    b = lax.axis_index("s")
    h = lax.axis_index("c")

    pltpu.sync_copy(lens_hbm, lens_v.at[pl.ds(0, 16)])
    L = lens_v[pl.ds(b, 16)][0]

    for fb in range(NFB):
        fs = pl.ds(fb * 16, 16)
        sum_v[fs] = jnp.zeros((16,), jnp.float32)
        max_v[fs] = jnp.full((16,), -1e30, jnp.float32)
        min_v[fs] = jnp.full((16,), 1e30, jnp.float32)

    Lc = jnp.minimum(L, S - 1)
    nt = (Lc + NR) // NR

    def _src(ci):
        return tokens_hbm.at[b, pl.ds(ci * NR, NR), pl.ds(h * FH, FH)]

    pltpu.async_copy(_src(0), buf.at[pl.ds(0, NR)], sem0)

    def chunk_body(ci, _):
        par = ci % 2
        nxt = (ci + 1) % 2

        @pl.when((ci + 1 < nt) & (nxt == 1))
        def _pre1():
            pltpu.async_copy(_src(ci + 1), buf.at[pl.ds(NR, NR)], sem1)

        @pl.when((ci + 1 < nt) & (nxt == 0))
        def _pre0():
            pltpu.async_copy(_src(ci + 1), buf.at[pl.ds(0, NR)], sem0)

        @pl.when(par == 0)
        def _wait0():
            pltpu.make_async_copy(_src(ci), buf.at[pl.ds(0, NR)], sem0).wait()

        @pl.when(par == 1)
        def _wait1():
            pltpu.make_async_copy(_src(ci), buf.at[pl.ds(NR, NR)], sem1).wait()

        base = par * NR
        lo = jnp.where(ci == 0, 1, 0)
        hi = jnp.minimum(NR, Lc + 1 - ci * NR)
        n = hi - lo
        ng = n // 8
        for fb2 in range(NFB // 2):
            fsa = pl.ds(fb2 * 32, 16)
            fsb = pl.ds(fb2 * 32 + 16, 16)

            def grp_body(g, carry):
                sma, mxa, mna, smb, mxb, mnb = carry
                r0 = base + lo + g * 8
                xa = [buf[r0 + j, fsa] for j in range(8)]
                xb = [buf[r0 + j, fsb] for j in range(8)]
                sma = sma + _tree3(xa, lambda u, v: u + v)
                mxa = jnp.maximum(mxa, _tree3(xa, jnp.maximum))
                mna = jnp.minimum(mna, _tree3(xa, jnp.minimum))
                smb = smb + _tree3(xb, lambda u, v: u + v)
                mxb = jnp.maximum(mxb, _tree3(xb, jnp.maximum))
                mnb = jnp.minimum(mnb, _tree3(xb, jnp.minimum))
                return sma, mxa, mna, smb, mxb, mnb

            def rem_body(r, carry):
                sma, mxa, mna, smb, mxb, mnb = carry
                ra = base + lo + ng * 8 + r
                xa = buf[ra, fsa]
                xb = buf[ra, fsb]
                return (sma + xa, jnp.maximum(mxa, xa), jnp.minimum(mna, xa),
                        smb + xb, jnp.maximum(mxb, xb), jnp.minimum(mnb, xb))

            carry = (sum_v[fsa], max_v[fsa], min_v[fsa],
                     sum_v[fsb], max_v[fsb], min_v[fsb])
            carry = lax.fori_loop(0, ng, grp_body, carry)
            carry = lax.fori_loop(0, n - ng * 8, rem_body, carry)
            (sum_v[fsa], max_v[fsa], min_v[fsa],
             sum_v[fsb], max_v[fsb], min_v[fsb]) = carry
        return 0

    lax.fori_loop(0, nt, chunk_body, 0)

    pltpu.sync_copy(sum_v, out_hbm.at[b, pl.ds(h * FH, FH)])
    pltpu.sync_copy(max_v, out_hbm.at[b, pl.ds(D + h * FH, FH)])
    pltpu.sync_copy(min_v, out_hbm.at[b, pl.ds(2 * D + h * FH, FH)])


_sc_pool = functools.partial(
    pl.kernel,
    out_type=jax.ShapeDtypeStruct((B, 3 * D), jnp.float32),
    mesh=plsc.VectorSubcoreMesh(core_axis_name="c", subcore_axis_name="s"),
    scratch_types=[
        pltpu.VMEM((32,), jnp.int32),
        pltpu.VMEM((2 * NR, FH), jnp.float32),
        pltpu.VMEM((FH,), jnp.float32),
        pltpu.VMEM((FH,), jnp.float32),
        pltpu.VMEM((FH,), jnp.float32),
        pltpu.SemaphoreType.DMA,
        pltpu.SemaphoreType.DMA,
    ],
)(_sc_pool_body)


def _attn_body(lens_ref, tok_ref, q_ref, out_ref, acc_s, l_s):
    b = pl.program_id(0)
    c = pl.program_id(1)
    L = lens_ref[b]

    @pl.when(c == 0)
    def _init():
        acc_s[...] = jnp.zeros_like(acc_s)
        l_s[...] = jnp.zeros_like(l_s)

    full = (c > 0) & (c * CS + (CS - 1) <= L)

    @pl.when(full)
    def _accum_full():
        x = tok_ref[0]
        s = jax.lax.dot_general(x, q_ref[...], (((1,), (1,)), ((), ())),
                                preferred_element_type=jnp.float32) * SCALE
        p = jnp.exp(s)
        l_s[...] += jnp.sum(p, axis=0)[:, None]
        acc_s[...] += jax.lax.dot_general(p, x, (((0,), (0,)), ((), ())),
                                          preferred_element_type=jnp.float32)

    @pl.when((c * CS <= L) & jnp.logical_not(full))
    def _accum_boundary():
        x = tok_ref[0]
        r = jax.lax.broadcasted_iota(jnp.int32, (CS, 1), 0) + c * CS
        valid = (r >= 1) & (r <= L)
        s = jax.lax.dot_general(x, q_ref[...], (((1,), (1,)), ((), ())),
                                preferred_element_type=jnp.float32) * SCALE
        p = jnp.where(valid, jnp.exp(s), 0.0)
        l_s[...] += jnp.sum(p, axis=0)[:, None]
        acc_s[...] += jax.lax.dot_general(p, x, (((0,), (0,)), ((), ())),
                                          preferred_element_type=jnp.float32)

    @pl.when(c == NC - 1)
    def _finalize():
        out_ref[0] = acc_s[...] / l_s[:, 0:1]


def _attn(lengths_i32, tokens, qpad):
    grid_spec = pltpu.PrefetchScalarGridSpec(
        num_scalar_prefetch=1,
        grid=(B, NC),
        in_specs=[
            pl.BlockSpec((1, CS, D),
                         lambda b, c, lens: (b, jnp.minimum(c, lens[b] // CS), 0)),
            pl.BlockSpec((QP, D), lambda b, c, lens: (0, 0)),
        ],
        out_specs=pl.BlockSpec((1, QP, D), lambda b, c, lens: (b, 0, 0)),
        scratch_shapes=[
            pltpu.VMEM((QP, D), jnp.float32),
            pltpu.VMEM((QP, 128), jnp.float32),
        ],
    )
    return pl.pallas_call(
        _attn_body,
        grid_spec=grid_spec,
        out_shape=jax.ShapeDtypeStruct((B, QP, D), jnp.float32),
    )(lengths_i32, tokens, qpad)


def _gelu_exact(x):
    return 0.5 * x * (1.0 + jax.lax.erf(x * (2.0 ** -0.5)))


def _mlp_body(x1_ref, x2_ref, lens_ref, last_ref, W1a_ref, b1a_ref, W1b_ref,
              b1b_ref, W2a_ref, b2a_ref, W2b_ref, b2b_ref, out_ref):
    L = lens_ref[...]
    lr = last_ref[:, 0, :]
    is_full = L == float(S)
    sum_p = x1_ref[:, 0:D] + jnp.where(is_full, lr, 0.0)
    max_p = x1_ref[:, D:2 * D]
    max_p = jnp.where(is_full, jnp.maximum(max_p, lr), max_p)
    min_p = x1_ref[:, 2 * D:3 * D]
    min_p = jnp.where(is_full, jnp.minimum(min_p, lr), min_p)
    x1 = jnp.concatenate([sum_p / L, max_p, min_p], axis=-1)

    h1 = _gelu_exact(
        jnp.dot(x1, W1a_ref[...], preferred_element_type=jnp.float32)
        + b1a_ref[...])
    o1 = jnp.dot(h1, W1b_ref[...], preferred_element_type=jnp.float32) + b1b_ref[...]
    h2 = _gelu_exact(
        jnp.dot(x2_ref[...], W2a_ref[...], preferred_element_type=jnp.float32)
        + b2a_ref[...])
    o2 = jnp.dot(h2, W2b_ref[...], preferred_element_type=jnp.float32) + b2b_ref[...]
    out_ref[:, 0:D] = o1
    out_ref[:, D:2 * D] = o2


def _mlp(x1raw, x2, lens_f, tokens, W1a, b1a, W1b, b1b, W2a, b2a, W2b, b2b):
    whole = lambda *s: pl.BlockSpec(s, lambda i: (0,) * len(s))
    return pl.pallas_call(
        _mlp_body,
        grid=(1,),
        in_specs=[
            whole(B, 3 * D),
            whole(B, (1 + M) * D),
            whole(B, 1),
            pl.BlockSpec((B, 8, D), lambda i: (0, S // 8, 0)),
            whole(3 * D, D), whole(1, D), whole(D, D), whole(1, D),
            whole((1 + M) * D, D), whole(1, D), whole(D, D), whole(1, D),
        ],
        out_specs=whole(B, 2 * D),
        out_shape=jax.ShapeDtypeStruct((B, 2 * D), jnp.float32),
    )(x1raw, x2, lens_f, tokens, W1a, b1a, W1b, b1b, W2a, b2a, W2b, b2b)


@jax.jit
def kernel(tokens, lengths, queries, W1a, b1a, W1b, b1b, W2a, b2a, W2b, b2b):
    lengths_i32 = lengths.astype(jnp.int32)
    qpad = jnp.zeros((QP, D), jnp.float32).at[:M].set(queries)
    pmp8 = _attn(lengths_i32, tokens, qpad)
    pooled_raw = _sc_pool(lengths_i32, tokens)
    clf = tokens[:, 0]
    x2 = jnp.concatenate([pmp8[:, :M, :].reshape(B, M * D), clf], axis=-1)
    lens_f = lengths_i32.astype(jnp.float32).reshape(B, 1)
    return _mlp(pooled_raw, x2, lens_f, tokens,
                W1a, b1a.reshape(1, D), W1b, b1b.reshape(1, D),
                W2a, b2a.reshape(1, D), W2b, b2b.reshape(1, D))


# speedup vs baseline: 1.0115x; 1.0115x over previous
import functools

import jax
import jax.numpy as jnp
from jax import lax
from jax.experimental import pallas as pl
from jax.experimental.pallas import tpu as pltpu
from jax.experimental.pallas import tpu_sc as plsc

B, S, D, M = 16, 4096, 768, 4
CS = 1024
NC = (S + 1 + CS - 1) // CS
QP = 8
SCALE = D ** -0.5

FH = D // 2
NFB = FH // 16
NR = 128


def _tree3(vals, op):
    while len(vals) > 1:
        vals = [op(vals[i], vals[i + 1]) for i in range(0, len(vals) - 1, 2)] \
            + ([vals[-1]] if len(vals) % 2 else [])
    return vals[0]


def _sc_pool_body(lens_hbm, tokens_hbm, out_hbm, lens_v, buf, sum_v, max_v,
                  min_v, sem0, sem1):
    b = lax.axis_index("s")
    h = lax.axis_index("c")

    pltpu.sync_copy(lens_hbm, lens_v.at[pl.ds(0, 16)])
    L = lens_v[pl.ds(b, 16)][0]

    for fb in range(NFB):
        fs = pl.ds(fb * 16, 16)
        sum_v[fs] = jnp.zeros((16,), jnp.float32)
        max_v[fs] = jnp.full((16,), -1e30, jnp.float32)
        min_v[fs] = jnp.full((16,), 1e30, jnp.float32)

    Lc = jnp.minimum(L, S - 1)
    nt = (Lc + NR) // NR

    def _src(ci):
        return tokens_hbm.at[b, pl.ds(ci * NR, NR), pl.ds(h * FH, FH)]

    pltpu.async_copy(_src(0), buf.at[pl.ds(0, NR)], sem0)

    def chunk_body(ci, _):
        par = ci % 2
        nxt = (ci + 1) % 2

        @pl.when((ci + 1 < nt) & (nxt == 1))
        def _pre1():
            pltpu.async_copy(_src(ci + 1), buf.at[pl.ds(NR, NR)], sem1)

        @pl.when((ci + 1 < nt) & (nxt == 0))
        def _pre0():
            pltpu.async_copy(_src(ci + 1), buf.at[pl.ds(0, NR)], sem0)

        @pl.when(par == 0)
        def _wait0():
            pltpu.make_async_copy(_src(ci), buf.at[pl.ds(0, NR)], sem0).wait()

        @pl.when(par == 1)
        def _wait1():
            pltpu.make_async_copy(_src(ci), buf.at[pl.ds(NR, NR)], sem1).wait()

        base = par * NR
        lo = jnp.where(ci == 0, 1, 0)
        hi = jnp.minimum(NR, Lc + 1 - ci * NR)
        n = hi - lo
        ng = n // 8
        for fb2 in range(NFB // 2):
            fsa = pl.ds(fb2 * 32, 16)
            fsb = pl.ds(fb2 * 32 + 16, 16)

            def grp_body(g, carry):
                sma, mxa, mna, smb, mxb, mnb = carry
                r0 = base + lo + g * 8
                xa = [buf[r0 + j, fsa] for j in range(8)]
                xb = [buf[r0 + j, fsb] for j in range(8)]
                sma = sma + _tree3(xa, lambda u, v: u + v)
                mxa = jnp.maximum(mxa, _tree3(xa, jnp.maximum))
                mna = jnp.minimum(mna, _tree3(xa, jnp.minimum))
                smb = smb + _tree3(xb, lambda u, v: u + v)
                mxb = jnp.maximum(mxb, _tree3(xb, jnp.maximum))
                mnb = jnp.minimum(mnb, _tree3(xb, jnp.minimum))
                return sma, mxa, mna, smb, mxb, mnb

            def rem_body(r, carry):
                sma, mxa, mna, smb, mxb, mnb = carry
                ra = base + lo + ng * 8 + r
                xa = buf[ra, fsa]
                xb = buf[ra, fsb]
                return (sma + xa, jnp.maximum(mxa, xa), jnp.minimum(mna, xa),
                        smb + xb, jnp.maximum(mxb, xb), jnp.minimum(mnb, xb))

            carry = (sum_v[fsa], max_v[fsa], min_v[fsa],
                     sum_v[fsb], max_v[fsb], min_v[fsb])
            carry = lax.fori_loop(0, ng, grp_body, carry)
            carry = lax.fori_loop(0, n - ng * 8, rem_body, carry)
            (sum_v[fsa], max_v[fsa], min_v[fsa],
             sum_v[fsb], max_v[fsb], min_v[fsb]) = carry
        return 0

    lax.fori_loop(0, nt, chunk_body, 0)

    pltpu.sync_copy(sum_v, out_hbm.at[b, pl.ds(h * FH, FH)])
    pltpu.sync_copy(max_v, out_hbm.at[b, pl.ds(D + h * FH, FH)])
    pltpu.sync_copy(min_v, out_hbm.at[b, pl.ds(2 * D + h * FH, FH)])


_sc_pool = functools.partial(
    pl.kernel,
    out_type=jax.ShapeDtypeStruct((B, 3 * D), jnp.float32),
    mesh=plsc.VectorSubcoreMesh(core_axis_name="c", subcore_axis_name="s"),
    scratch_types=[
        pltpu.VMEM((32,), jnp.int32),
        pltpu.VMEM((2 * NR, FH), jnp.float32),
        pltpu.VMEM((FH,), jnp.float32),
        pltpu.VMEM((FH,), jnp.float32),
        pltpu.VMEM((FH,), jnp.float32),
        pltpu.SemaphoreType.DMA,
        pltpu.SemaphoreType.DMA,
    ],
)(_sc_pool_body)


def _attn_body(lens_ref, tok_ref, q_ref, out_ref, acc_s, l_s):
    b = pl.program_id(0)
    c = pl.program_id(1)
    L = lens_ref[b]

    @pl.when(c == 0)
    def _init():
        acc_s[...] = jnp.zeros_like(acc_s)
        l_s[...] = jnp.zeros_like(l_s)

    full = (c > 0) & (c * CS + (CS - 1) <= L)

    @pl.when(full)
    def _accum_full():
        xb = tok_ref[0].astype(jnp.bfloat16)
        s = jax.lax.dot_general(xb, q_ref[...], (((1,), (1,)), ((), ())),
                                preferred_element_type=jnp.float32) * SCALE
        p = jnp.exp(s)
        l_s[...] += jnp.sum(p, axis=0)[:, None]
        acc_s[...] += jax.lax.dot_general(p.astype(jnp.bfloat16), xb,
                                          (((0,), (0,)), ((), ())),
                                          preferred_element_type=jnp.float32)

    @pl.when((c * CS <= L) & jnp.logical_not(full))
    def _accum_boundary():
        xb = tok_ref[0].astype(jnp.bfloat16)
        r = jax.lax.broadcasted_iota(jnp.int32, (CS, 1), 0) + c * CS
        valid = (r >= 1) & (r <= L)
        s = jax.lax.dot_general(xb, q_ref[...], (((1,), (1,)), ((), ())),
                                preferred_element_type=jnp.float32) * SCALE
        p = jnp.where(valid, jnp.exp(s), 0.0)
        l_s[...] += jnp.sum(p, axis=0)[:, None]
        acc_s[...] += jax.lax.dot_general(p.astype(jnp.bfloat16), xb,
                                          (((0,), (0,)), ((), ())),
                                          preferred_element_type=jnp.float32)

    @pl.when(c == NC - 1)
    def _finalize():
        out_ref[0] = acc_s[...] / l_s[:, 0:1]


def _attn(lengths_i32, tokens, qpad):
    grid_spec = pltpu.PrefetchScalarGridSpec(
        num_scalar_prefetch=1,
        grid=(B, NC),
        in_specs=[
            pl.BlockSpec((1, CS, D),
                         lambda b, c, lens: (b, jnp.minimum(c, lens[b] // CS), 0)),
            pl.BlockSpec((QP, D), lambda b, c, lens: (0, 0)),
        ],
        out_specs=pl.BlockSpec((1, QP, D), lambda b, c, lens: (b, 0, 0)),
        scratch_shapes=[
            pltpu.VMEM((QP, D), jnp.float32),
            pltpu.VMEM((QP, 128), jnp.float32),
        ],
    )
    return pl.pallas_call(
        _attn_body,
        grid_spec=grid_spec,
        out_shape=jax.ShapeDtypeStruct((B, QP, D), jnp.float32),
    )(lengths_i32, tokens, qpad)


def _gelu_exact(x):
    return 0.5 * x * (1.0 + jax.lax.erf(x * (2.0 ** -0.5)))


def _mlp_body(x1_ref, x2_ref, lens_ref, last_ref, W1a_ref, b1a_ref, W1b_ref,
              b1b_ref, W2a_ref, b2a_ref, W2b_ref, b2b_ref, out_ref):
    L = lens_ref[...]
    lr = last_ref[:, 0, :]
    is_full = L == float(S)
    sum_p = x1_ref[:, 0:D] + jnp.where(is_full, lr, 0.0)
    max_p = x1_ref[:, D:2 * D]
    max_p = jnp.where(is_full, jnp.maximum(max_p, lr), max_p)
    min_p = x1_ref[:, 2 * D:3 * D]
    min_p = jnp.where(is_full, jnp.minimum(min_p, lr), min_p)
    x1 = jnp.concatenate([sum_p / L, max_p, min_p], axis=-1)

    h1 = _gelu_exact(
        jnp.dot(x1, W1a_ref[...], preferred_element_type=jnp.float32)
        + b1a_ref[...])
    o1 = jnp.dot(h1, W1b_ref[...], preferred_element_type=jnp.float32) + b1b_ref[...]
    h2 = _gelu_exact(
        jnp.dot(x2_ref[...], W2a_ref[...], preferred_element_type=jnp.float32)
        + b2a_ref[...])
    o2 = jnp.dot(h2, W2b_ref[...], preferred_element_type=jnp.float32) + b2b_ref[...]
    out_ref[:, 0:D] = o1
    out_ref[:, D:2 * D] = o2


def _mlp(x1raw, x2, lens_f, tokens, W1a, b1a, W1b, b1b, W2a, b2a, W2b, b2b):
    whole = lambda *s: pl.BlockSpec(s, lambda i: (0,) * len(s))
    return pl.pallas_call(
        _mlp_body,
        grid=(1,),
        in_specs=[
            whole(B, 3 * D),
            whole(B, (1 + M) * D),
            whole(B, 1),
            pl.BlockSpec((B, 8, D), lambda i: (0, S // 8, 0)),
            whole(3 * D, D), whole(1, D), whole(D, D), whole(1, D),
            whole((1 + M) * D, D), whole(1, D), whole(D, D), whole(1, D),
        ],
        out_specs=whole(B, 2 * D),
        out_shape=jax.ShapeDtypeStruct((B, 2 * D), jnp.float32),
    )(x1raw, x2, lens_f, tokens, W1a, b1a, W1b, b1b, W2a, b2a, W2b, b2b)


@jax.jit
def kernel(tokens, lengths, queries, W1a, b1a, W1b, b1b, W2a, b2a, W2b, b2b):
    lengths_i32 = lengths.astype(jnp.int32)
    qpad = jnp.zeros((QP, D), jnp.bfloat16).at[:M].set(
        queries.astype(jnp.bfloat16))
    pmp8 = _attn(lengths_i32, tokens, qpad)
    pooled_raw = _sc_pool(lengths_i32, tokens)
    clf = tokens[:, 0]
    x2 = jnp.concatenate([pmp8[:, :M, :].reshape(B, M * D), clf], axis=-1)
    lens_f = lengths_i32.astype(jnp.float32).reshape(B, 1)
    return _mlp(pooled_raw, x2, lens_f, tokens,
                W1a, b1a.reshape(1, D), W1b, b1b.reshape(1, D),
                W2a, b2a.reshape(1, D), W2b, b2b.reshape(1, D))


# speedup vs baseline: 1.0140x; 1.0025x over previous
import functools

import jax
import jax.numpy as jnp
from jax import lax
from jax.experimental import pallas as pl
from jax.experimental.pallas import tpu as pltpu
from jax.experimental.pallas import tpu_sc as plsc

B, S, D, M = 16, 4096, 768, 4
CS = 1024
NC = (S + 1 + CS - 1) // CS
QP = 8
SCALE = D ** -0.5

FH = D // 2
NFB = FH // 16
NR = 128


def _tree3(vals, op):
    while len(vals) > 1:
        vals = [op(vals[i], vals[i + 1]) for i in range(0, len(vals) - 1, 2)] \
            + ([vals[-1]] if len(vals) % 2 else [])
    return vals[0]


def _sc_pool_body(lens_hbm, tokens_hbm, out_hbm, lens_v, buf, sum_v, max_v,
                  min_v, stage, sem0, sem1):
    c = lax.axis_index("c")
    s = lax.axis_index("s")
    h = s % 2
    stripe = s // 2
    fo = h * FH

    pltpu.sync_copy(lens_hbm, lens_v.at[pl.ds(0, 16)])

    Lc = []
    pref = [jnp.int32(0)]
    for g in range(8):
        Lg = lens_v[pl.ds(c * 8 + g, 16)][0]
        Lcg = jnp.minimum(Lg, S - 1)
        Lc.append(Lcg)
        pref.append(pref[-1] + (Lcg + NR) // NR)
    T = pref[8]

    def _sel(vals, idx):
        r = vals[0]
        for g in range(1, 8):
            r = jnp.where(idx == g, vals[g], r)
        return r

    def _decode(k):
        g = jnp.int32(0)
        for gg in range(1, 8):
            g = g + (pref[gg] <= k).astype(jnp.int32)
        ci = k - _sel(pref[:8], g)
        return g, ci, _sel(Lc, g)

    def _src(k):
        g, ci, _ = _decode(k)
        return tokens_hbm.at[c * 8 + g, pl.ds(ci * NR, NR), pl.ds(fo, FH)]

    def _init(i, _):
        fs = pl.ds(i * 16, 16)
        sum_v[fs] = jnp.zeros((16,), jnp.float32)
        max_v[fs] = jnp.full((16,), -1e30, jnp.float32)
        min_v[fs] = jnp.full((16,), 1e30, jnp.float32)
        return 0

    lax.fori_loop(0, 8 * FH // 16, _init, 0)

    @pl.when(stripe < T)
    def _pro():
        pltpu.async_copy(_src(stripe), buf.at[pl.ds(0, NR)], sem0)

    nit = (T - stripe + 7) // 8

    def body(i, _):
        k = stripe + 8 * i
        par = i % 2

        @pl.when((k + 8 < T) & (par == 0))
        def _p1():
            pltpu.async_copy(_src(k + 8), buf.at[pl.ds(NR, NR)], sem1)

        @pl.when((k + 8 < T) & (par == 1))
        def _p0():
            pltpu.async_copy(_src(k + 8), buf.at[pl.ds(0, NR)], sem0)

        @pl.when(par == 0)
        def _w0():
            pltpu.make_async_copy(_src(k), buf.at[pl.ds(0, NR)], sem0).wait()

        @pl.when(par == 1)
        def _w1():
            pltpu.make_async_copy(_src(k), buf.at[pl.ds(NR, NR)], sem1).wait()

        g, ci, lc = _decode(k)
        base = par * NR
        po = g * FH
        lo = jnp.where(ci == 0, 1, 0)
        hi = jnp.minimum(NR, lc + 1 - ci * NR)
        n = hi - lo
        ng = n // 8
        for fb2 in range(NFB // 2):
            fsa = pl.ds(po + fb2 * 32, 16)
            fsb = pl.ds(po + fb2 * 32 + 16, 16)
            bfa = pl.ds(fb2 * 32, 16)
            bfb = pl.ds(fb2 * 32 + 16, 16)

            def grp_body(gi, carry):
                sma, mxa, mna, smb, mxb, mnb = carry
                r0 = base + lo + gi * 8
                xa = [buf[r0 + j, bfa] for j in range(8)]
                xb = [buf[r0 + j, bfb] for j in range(8)]
                sma = sma + _tree3(xa, lambda u, v: u + v)
                mxa = jnp.maximum(mxa, _tree3(xa, jnp.maximum))
                mna = jnp.minimum(mna, _tree3(xa, jnp.minimum))
                smb = smb + _tree3(xb, lambda u, v: u + v)
                mxb = jnp.maximum(mxb, _tree3(xb, jnp.maximum))
                mnb = jnp.minimum(mnb, _tree3(xb, jnp.minimum))
                return sma, mxa, mna, smb, mxb, mnb

            def rem_body(r, carry):
                sma, mxa, mna, smb, mxb, mnb = carry
                ra = base + lo + ng * 8 + r
                xa = buf[ra, bfa]
                xb = buf[ra, bfb]
                return (sma + xa, jnp.maximum(mxa, xa), jnp.minimum(mna, xa),
                        smb + xb, jnp.maximum(mxb, xb), jnp.minimum(mnb, xb))

            carry = (sum_v[fsa], max_v[fsa], min_v[fsa],
                     sum_v[fsb], max_v[fsb], min_v[fsb])
            carry = lax.fori_loop(0, ng, grp_body, carry)
            carry = lax.fori_loop(0, n - ng * 8, rem_body, carry)
            (sum_v[fsa], max_v[fsa], min_v[fsa],
             sum_v[fsb], max_v[fsb], min_v[fsb]) = carry
        return 0

    lax.fori_loop(0, nit, body, 0)

    PS = 8 * FH
    pltpu.sync_copy(sum_v, stage.at[pl.ds((s * 3 + 0) * PS, PS)])
    pltpu.sync_copy(max_v, stage.at[pl.ds((s * 3 + 1) * PS, PS)])
    pltpu.sync_copy(min_v, stage.at[pl.ds((s * 3 + 2) * PS, PS)])
    plsc.subcore_barrier()

    gm = s // 2
    hm = s % 2
    for q, opf in ((0, lambda u, v: u + v), (1, jnp.maximum),
                   (2, jnp.minimum)):
        for t in range(8):
            off = ((hm + 2 * t) * 3 + q) * PS + gm * FH
            pltpu.sync_copy(stage.at[pl.ds(off, FH)],
                            min_v.at[pl.ds(t * FH, FH)])
        for fb in range(NFB):
            acc = min_v[pl.ds(fb * 16, 16)]
            for t in range(1, 8):
                acc = opf(acc, min_v[pl.ds(t * FH + fb * 16, 16)])
            max_v[pl.ds(fb * 16, 16)] = acc
        pltpu.sync_copy(
            max_v.at[pl.ds(0, FH)],
            out_hbm.at[c * 8 + gm, pl.ds(q * D + hm * FH, FH)])


_sc_pool = functools.partial(
    pl.kernel,
    out_type=jax.ShapeDtypeStruct((B, 3 * D), jnp.float32),
    mesh=plsc.VectorSubcoreMesh(core_axis_name="c", subcore_axis_name="s"),
    scratch_types=[
        pltpu.VMEM((32,), jnp.int32),
        pltpu.VMEM((2 * NR, FH), jnp.float32),
        pltpu.VMEM((8 * FH,), jnp.float32),
        pltpu.VMEM((8 * FH,), jnp.float32),
        pltpu.VMEM((8 * FH,), jnp.float32),
        pltpu.VMEM_SHARED((16 * 3 * 8 * FH,), jnp.float32),
        pltpu.SemaphoreType.DMA,
        pltpu.SemaphoreType.DMA,
    ],
)(_sc_pool_body)


def _attn_body(lens_ref, tok_ref, q_ref, out_ref, acc_s, l_s):
    b = pl.program_id(0)
    c = pl.program_id(1)
    L = lens_ref[b]

    @pl.when(c == 0)
    def _init():
        acc_s[...] = jnp.zeros_like(acc_s)
        l_s[...] = jnp.zeros_like(l_s)

    full = (c > 0) & (c * CS + (CS - 1) <= L)

    @pl.when(full)
    def _accum_full():
        xb = tok_ref[0].astype(jnp.bfloat16)
        s = jax.lax.dot_general(xb, q_ref[...], (((1,), (1,)), ((), ())),
                                preferred_element_type=jnp.float32) * SCALE
        p = jnp.exp(s)
        l_s[...] += jnp.sum(p, axis=0)[:, None]
        acc_s[...] += jax.lax.dot_general(p.astype(jnp.bfloat16), xb,
                                          (((0,), (0,)), ((), ())),
                                          preferred_element_type=jnp.float32)

    @pl.when((c * CS <= L) & jnp.logical_not(full))
    def _accum_boundary():
        xb = tok_ref[0].astype(jnp.bfloat16)
        r = jax.lax.broadcasted_iota(jnp.int32, (CS, 1), 0) + c * CS
        valid = (r >= 1) & (r <= L)
        s = jax.lax.dot_general(xb, q_ref[...], (((1,), (1,)), ((), ())),
                                preferred_element_type=jnp.float32) * SCALE
        p = jnp.where(valid, jnp.exp(s), 0.0)
        l_s[...] += jnp.sum(p, axis=0)[:, None]
        acc_s[...] += jax.lax.dot_general(p.astype(jnp.bfloat16), xb,
                                          (((0,), (0,)), ((), ())),
                                          preferred_element_type=jnp.float32)

    @pl.when(c == NC - 1)
    def _finalize():
        out_ref[0] = acc_s[...] / l_s[:, 0:1]


def _attn(lengths_i32, tokens, qpad):
    grid_spec = pltpu.PrefetchScalarGridSpec(
        num_scalar_prefetch=1,
        grid=(B, NC),
        in_specs=[
            pl.BlockSpec((1, CS, D),
                         lambda b, c, lens: (b, jnp.minimum(c, lens[b] // CS), 0)),
            pl.BlockSpec((QP, D), lambda b, c, lens: (0, 0)),
        ],
        out_specs=pl.BlockSpec((1, QP, D), lambda b, c, lens: (b, 0, 0)),
        scratch_shapes=[
            pltpu.VMEM((QP, D), jnp.float32),
            pltpu.VMEM((QP, 128), jnp.float32),
        ],
    )
    return pl.pallas_call(
        _attn_body,
        grid_spec=grid_spec,
        out_shape=jax.ShapeDtypeStruct((B, QP, D), jnp.float32),
    )(lengths_i32, tokens, qpad)


def _gelu_exact(x):
    return 0.5 * x * (1.0 + jax.lax.erf(x * (2.0 ** -0.5)))


def _mlp_body(x1_ref, x2_ref, lens_ref, last_ref, W1a_ref, b1a_ref, W1b_ref,
              b1b_ref, W2a_ref, b2a_ref, W2b_ref, b2b_ref, out_ref):
    L = lens_ref[...]
    lr = last_ref[:, 0, :]
    is_full = L == float(S)
    sum_p = x1_ref[:, 0:D] + jnp.where(is_full, lr, 0.0)
    max_p = x1_ref[:, D:2 * D]
    max_p = jnp.where(is_full, jnp.maximum(max_p, lr), max_p)
    min_p = x1_ref[:, 2 * D:3 * D]
    min_p = jnp.where(is_full, jnp.minimum(min_p, lr), min_p)
    x1 = jnp.concatenate([sum_p / L, max_p, min_p], axis=-1)

    h1 = _gelu_exact(
        jnp.dot(x1, W1a_ref[...], preferred_element_type=jnp.float32)
        + b1a_ref[...])
    o1 = jnp.dot(h1, W1b_ref[...], preferred_element_type=jnp.float32) + b1b_ref[...]
    h2 = _gelu_exact(
        jnp.dot(x2_ref[...], W2a_ref[...], preferred_element_type=jnp.float32)
        + b2a_ref[...])
    o2 = jnp.dot(h2, W2b_ref[...], preferred_element_type=jnp.float32) + b2b_ref[...]
    out_ref[:, 0:D] = o1
    out_ref[:, D:2 * D] = o2


def _mlp(x1raw, x2, lens_f, tokens, W1a, b1a, W1b, b1b, W2a, b2a, W2b, b2b):
    whole = lambda *s: pl.BlockSpec(s, lambda i: (0,) * len(s))
    return pl.pallas_call(
        _mlp_body,
        grid=(1,),
        in_specs=[
            whole(B, 3 * D),
            whole(B, (1 + M) * D),
            whole(B, 1),
            pl.BlockSpec((B, 8, D), lambda i: (0, S // 8, 0)),
            whole(3 * D, D), whole(1, D), whole(D, D), whole(1, D),
            whole((1 + M) * D, D), whole(1, D), whole(D, D), whole(1, D),
        ],
        out_specs=whole(B, 2 * D),
        out_shape=jax.ShapeDtypeStruct((B, 2 * D), jnp.float32),
    )(x1raw, x2, lens_f, tokens, W1a, b1a, W1b, b1b, W2a, b2a, W2b, b2b)


@jax.jit
def kernel(tokens, lengths, queries, W1a, b1a, W1b, b1b, W2a, b2a, W2b, b2b):
    lengths_i32 = lengths.astype(jnp.int32)
    qpad = jnp.zeros((QP, D), jnp.bfloat16).at[:M].set(
        queries.astype(jnp.bfloat16))
    pmp8 = _attn(lengths_i32, tokens, qpad)
    pooled_raw = _sc_pool(lengths_i32, tokens)
    clf = tokens[:, 0]
    x2 = jnp.concatenate([pmp8[:, :M, :].reshape(B, M * D), clf], axis=-1)
    lens_f = lengths_i32.astype(jnp.float32).reshape(B, 1)
    return _mlp(pooled_raw, x2, lens_f, tokens,
                W1a, b1a.reshape(1, D), W1b, b1b.reshape(1, D),
                W2a, b2a.reshape(1, D), W2b, b2b.reshape(1, D))
